# Initial kernel scaffold; baseline (speedup 1.0000x reference)
import jax
import jax.numpy as jnp
from jax.experimental import pallas as pl


def kernel(X, Y, gate_theta):
    raise NotImplementedError("write your pallas kernel here")


# speedup vs baseline: 3.6007x; 3.6007x over previous
import functools

import jax
import jax.numpy as jnp
from jax import lax
from jax.experimental import pallas as pl
from jax.experimental.pallas import tpu as pltpu
from jax.experimental.pallas import tpu_sc as plsc

_NUM_ROWS = 10000
_H = 128
_B = 320000

_NC = 2
_NS = 16
_NW = _NC * _NS
_BPW = _B // _NW
_CHUNK = 80
_NCHUNKS = _BPW // _CHUNK


def _sigmoid_body(t_ref, o_ref):
    o_ref[...] = jax.nn.sigmoid(t_ref[...])


def _sigmoid_table(gate_theta):
    return pl.pallas_call(
        _sigmoid_body,
        out_shape=jax.ShapeDtypeStruct((_NUM_ROWS, _H), jnp.float32),
    )(gate_theta)


def _gather(sig_table, idx):
    mesh = plsc.VectorSubcoreMesh(core_axis_name="c", subcore_axis_name="s")

    @functools.partial(
        pl.kernel,
        mesh=mesh,
        out_type=jax.ShapeDtypeStruct((_B, _H), jnp.float32),
        scratch_types=[
            pltpu.VMEM((_BPW,), jnp.int32),
            pltpu.VMEM((_CHUNK, _H), jnp.float32),
            pltpu.SemaphoreType.DMA,
        ],
    )
    def k(table_hbm, idx_hbm, out_hbm, idx_v, rows_v, sem):
        wid = lax.axis_index("s") * _NC + lax.axis_index("c")
        base = wid * _BPW
        pltpu.sync_copy(idx_hbm.at[pl.ds(base, _BPW)], idx_v)

        def body(i, carry):
            off = i * _CHUNK
            pltpu.async_copy(
                table_hbm.at[idx_v.at[pl.ds(off, _CHUNK)]], rows_v, sem
            ).wait()
            pltpu.sync_copy(rows_v, out_hbm.at[pl.ds(base + off, _CHUNK)])
            return carry

        lax.fori_loop(0, _NCHUNKS, body, 0)

    return k(sig_table, idx)


def kernel(X, Y, gate_theta):
    sig_table = _sigmoid_table(gate_theta)
    idx = Y.astype(jnp.int32)
    return _gather(sig_table, idx)


# speedup vs baseline: 6.3614x; 1.7667x over previous
import functools

import jax
import jax.numpy as jnp
from jax import lax
from jax.experimental import pallas as pl
from jax.experimental.pallas import tpu as pltpu
from jax.experimental.pallas import tpu_sc as plsc

_NUM_ROWS = 10000
_H = 128
_B = 320000

_NC = 2
_NS = 16
_NW = _NC * _NS
_BPW = _B // _NW
_CHUNK = 80
_NCHUNKS = _BPW // _CHUNK
_NPAIRS = _NCHUNKS // 2
_RPT = 624
_RPT_REM = _NUM_ROWS - _NS * _RPT


def _sigmoid_body(t_ref, o_ref):
    o_ref[...] = jax.nn.sigmoid(t_ref[...])


def _sigmoid_table(gate_theta):
    return pl.pallas_call(
        _sigmoid_body,
        out_shape=jax.ShapeDtypeStruct((_NUM_ROWS, _H), jnp.float32),
    )(gate_theta)


def _gather(sig_table, idx):
    mesh = plsc.VectorSubcoreMesh(core_axis_name="c", subcore_axis_name="s")

    @functools.partial(
        pl.kernel,
        mesh=mesh,
        out_type=jax.ShapeDtypeStruct((_B, _H), jnp.float32),
        scratch_types=[
            pltpu.VMEM((_BPW,), jnp.int32),
            pltpu.VMEM((_CHUNK, _H), jnp.float32),
            pltpu.VMEM((_CHUNK, _H), jnp.float32),
            pltpu.VMEM_SHARED((_NUM_ROWS, _H), jnp.float32),
            pltpu.SemaphoreType.DMA,
            pltpu.SemaphoreType.DMA,
            pltpu.SemaphoreType.DMA,
            pltpu.SemaphoreType.DMA,
        ],
    )
    def k(table_hbm, idx_hbm, out_hbm, idx_v, rows_a, rows_b, tbl_sh,
          gsem_a, gsem_b, ssem_a, ssem_b):
        c = lax.axis_index("c")
        s = lax.axis_index("s")
        wid = s * _NC + c
        base = wid * _BPW

        pltpu.sync_copy(table_hbm.at[pl.ds(s * _RPT, _RPT)],
                        tbl_sh.at[pl.ds(s * _RPT, _RPT)])

        @pl.when(s == 0)
        def _stage_tail():
            pltpu.sync_copy(table_hbm.at[pl.ds(_NS * _RPT, _RPT_REM)],
                            tbl_sh.at[pl.ds(_NS * _RPT, _RPT_REM)])
        pltpu.sync_copy(idx_hbm.at[pl.ds(base, _BPW)], idx_v)
        plsc.subcore_barrier()

        def body(j, carry):
            o0 = (2 * j) * _CHUNK
            o1 = (2 * j + 1) * _CHUNK
            cp_a = pltpu.async_copy(
                tbl_sh.at[idx_v.at[pl.ds(o0, _CHUNK)]], rows_a, gsem_a)
            cp_b = pltpu.async_copy(
                tbl_sh.at[idx_v.at[pl.ds(o1, _CHUNK)]], rows_b, gsem_b)
            cp_a.wait()
            st_a = pltpu.async_copy(
                rows_a, out_hbm.at[pl.ds(base + o0, _CHUNK)], ssem_a)
            cp_b.wait()
            st_b = pltpu.async_copy(
                rows_b, out_hbm.at[pl.ds(base + o1, _CHUNK)], ssem_b)
            st_a.wait()
            st_b.wait()
            return carry

        lax.fori_loop(0, _NPAIRS, body, 0)

        ot = (_NCHUNKS - 1) * _CHUNK
        pltpu.async_copy(
            tbl_sh.at[idx_v.at[pl.ds(ot, _CHUNK)]], rows_a, gsem_a).wait()
        pltpu.sync_copy(rows_a, out_hbm.at[pl.ds(base + ot, _CHUNK)])

    return k(sig_table, idx)


def kernel(X, Y, gate_theta):
    sig_table = _sigmoid_table(gate_theta)
    idx = Y.astype(jnp.int32)
    return _gather(sig_table, idx)


# speedup vs baseline: 9.0070x; 1.4159x over previous
import functools

import jax
import jax.numpy as jnp
from jax import lax
from jax.experimental import pallas as pl
from jax.experimental.pallas import tpu as pltpu
from jax.experimental.pallas import tpu_sc as plsc

_NUM_ROWS = 10000
_H = 128
_B = 320000

_NC = 2
_NS = 16
_NW = _NC * _NS
_BPW = _B // _NW
_CHUNK = 80
_NCHUNKS = _BPW // _CHUNK
_NBUF = 4
_NGROUPS = _NCHUNKS // _NBUF
_RPT = 624
_RPT_REM = _NUM_ROWS - _NS * _RPT


def _sigmoid_body(t_ref, o_ref):
    o_ref[...] = jax.nn.sigmoid(t_ref[...])


def _sigmoid_table(gate_theta):
    return pl.pallas_call(
        _sigmoid_body,
        out_shape=jax.ShapeDtypeStruct((_NUM_ROWS, _H), jnp.float32),
    )(gate_theta)


def _gather(sig_table, idx):
    mesh = plsc.VectorSubcoreMesh(core_axis_name="c", subcore_axis_name="s")

    @functools.partial(
        pl.kernel,
        mesh=mesh,
        out_type=jax.ShapeDtypeStruct((_B, _H), jnp.float32),
        scratch_types=(
            [pltpu.VMEM((_BPW,), jnp.int32)]
            + [pltpu.VMEM((_CHUNK, _H), jnp.float32)] * _NBUF
            + [pltpu.VMEM_SHARED((_NUM_ROWS, _H), jnp.float32)]
            + [pltpu.SemaphoreType.DMA] * (2 * _NBUF)
        ),
    )
    def k(table_hbm, idx_hbm, out_hbm, idx_v, r0, r1, r2, r3, tbl_sh,
          g0, g1, g2, g3, s0, s1, s2, s3):
        bufs = [r0, r1, r2, r3]
        gsems = [g0, g1, g2, g3]
        ssems = [s0, s1, s2, s3]
        c = lax.axis_index("c")
        s = lax.axis_index("s")
        wid = s * _NC + c
        base = wid * _BPW

        pltpu.sync_copy(table_hbm.at[pl.ds(s * _RPT, _RPT)],
                        tbl_sh.at[pl.ds(s * _RPT, _RPT)])

        @pl.when(s == 0)
        def _stage_tail():
            pltpu.sync_copy(table_hbm.at[pl.ds(_NS * _RPT, _RPT_REM)],
                            tbl_sh.at[pl.ds(_NS * _RPT, _RPT_REM)])
        pltpu.sync_copy(idx_hbm.at[pl.ds(base, _BPW)], idx_v)
        plsc.subcore_barrier()

        def gather(off, b):
            return pltpu.async_copy(
                tbl_sh.at[idx_v.at[pl.ds(off, _CHUNK)]], bufs[b], gsems[b])

        def store(off, b):
            return pltpu.async_copy(
                bufs[b], out_hbm.at[pl.ds(base + off, _CHUNK)], ssems[b])

        def store_wait(b):
            pltpu.make_async_copy(
                bufs[b], out_hbm.at[pl.ds(base, _CHUNK)], ssems[b]).wait()

        prime = [gather(b * _CHUNK, b) for b in range(_NBUF)]
        for b in range(_NBUF):
            prime[b].wait()
            store(b * _CHUNK, b)

        def body(j, carry):
            handles = []
            for b in range(_NBUF):
                off = (_NBUF * j + b) * _CHUNK
                store_wait(b)
                handles.append(gather(off, b))
            for b in range(_NBUF):
                off = (_NBUF * j + b) * _CHUNK
                handles[b].wait()
                store(off, b)
            return carry

        lax.fori_loop(1, _NGROUPS, body, 0)

        for b in range(_NBUF):
            store_wait(b)

        ot = (_NCHUNKS - 1) * _CHUNK
        gather(ot, 0).wait()
        pltpu.sync_copy(bufs[0], out_hbm.at[pl.ds(base + ot, _CHUNK)])

    return k(sig_table, idx)


def kernel(X, Y, gate_theta):
    sig_table = _sigmoid_table(gate_theta)
    idx = Y.astype(jnp.int32)
    return _gather(sig_table, idx)


# speedup vs baseline: 9.0731x; 1.0073x over previous
import functools

import jax
import jax.numpy as jnp
from jax import lax
from jax.experimental import pallas as pl
from jax.experimental.pallas import tpu as pltpu
from jax.experimental.pallas import tpu_sc as plsc

_NUM_ROWS = 10000
_H = 128
_B = 320000

_NC = 2
_NS = 16
_NW = _NC * _NS
_BPW = _B // _NW
_CHUNK = 64
_NFULL = _BPW // _CHUNK
_TAIL = _BPW - _NFULL * _CHUNK
_NBUF = 5
_NGROUPS = _NFULL // _NBUF
_NLEFT = _NFULL - _NGROUPS * _NBUF
_RPT = 624
_RPT_REM = _NUM_ROWS - _NS * _RPT


def _sigmoid_body(t_ref, o_ref):
    o_ref[...] = jax.nn.sigmoid(t_ref[...])


def _sigmoid_table(gate_theta):
    return pl.pallas_call(
        _sigmoid_body,
        out_shape=jax.ShapeDtypeStruct((_NUM_ROWS, _H), jnp.float32),
    )(gate_theta)


def _gather(sig_table, idx):
    mesh = plsc.VectorSubcoreMesh(core_axis_name="c", subcore_axis_name="s")

    @functools.partial(
        pl.kernel,
        mesh=mesh,
        out_type=jax.ShapeDtypeStruct((_B, _H), jnp.float32),
        scratch_types=(
            [pltpu.VMEM((_BPW,), jnp.int32)]
            + [pltpu.VMEM((_CHUNK, _H), jnp.float32)] * _NBUF
            + [pltpu.VMEM_SHARED((_NUM_ROWS, _H), jnp.float32)]
            + [pltpu.SemaphoreType.DMA] * (2 * _NBUF)
        ),
    )
    def k(table_hbm, idx_hbm, out_hbm, *scratch):
        idx_v = scratch[0]
        bufs = list(scratch[1:1 + _NBUF])
        tbl_sh = scratch[1 + _NBUF]
        gsems = list(scratch[2 + _NBUF:2 + 2 * _NBUF])
        ssems = list(scratch[2 + 2 * _NBUF:2 + 3 * _NBUF])
        c = lax.axis_index("c")
        s = lax.axis_index("s")
        wid = s * _NC + c
        base = wid * _BPW

        pltpu.sync_copy(table_hbm.at[pl.ds(s * _RPT, _RPT)],
                        tbl_sh.at[pl.ds(s * _RPT, _RPT)])

        @pl.when(s == 0)
        def _stage_tail():
            pltpu.sync_copy(table_hbm.at[pl.ds(_NS * _RPT, _RPT_REM)],
                            tbl_sh.at[pl.ds(_NS * _RPT, _RPT_REM)])
        pltpu.sync_copy(idx_hbm.at[pl.ds(base, _BPW)], idx_v)
        plsc.subcore_barrier()

        def gather(off, b):
            return pltpu.async_copy(
                tbl_sh.at[idx_v.at[pl.ds(off, _CHUNK)]], bufs[b], gsems[b])

        def store(off, b):
            return pltpu.async_copy(
                bufs[b], out_hbm.at[pl.ds(base + off, _CHUNK)], ssems[b])

        def store_wait(b):
            pltpu.make_async_copy(
                bufs[b], out_hbm.at[pl.ds(base, _CHUNK)], ssems[b]).wait()

        prime = [gather(b * _CHUNK, b) for b in range(_NBUF)]
        for b in range(_NBUF):
            prime[b].wait()
            store(b * _CHUNK, b)

        def body(j, carry):
            handles = []
            for b in range(_NBUF):
                off = (_NBUF * j + b) * _CHUNK
                store_wait(b)
                handles.append(gather(off, b))
            for b in range(_NBUF):
                off = (_NBUF * j + b) * _CHUNK
                handles[b].wait()
                store(off, b)
            return carry

        lax.fori_loop(1, _NGROUPS, body, 0)

        left = []
        for b in range(_NLEFT):
            off = (_NGROUPS * _NBUF + b) * _CHUNK
            store_wait(b)
            left.append(gather(off, b))
        for b in range(_NLEFT):
            off = (_NGROUPS * _NBUF + b) * _CHUNK
            left[b].wait()
            store(off, b)

        if _TAIL:
            tb = _NLEFT
            ot = _NFULL * _CHUNK
            store_wait(tb)
            pltpu.async_copy(
                tbl_sh.at[idx_v.at[pl.ds(ot, _TAIL)]],
                bufs[tb].at[pl.ds(0, _TAIL)], gsems[tb]).wait()
            pltpu.sync_copy(bufs[tb].at[pl.ds(0, _TAIL)],
                            out_hbm.at[pl.ds(base + ot, _TAIL)])

        for b in range(_NBUF):
            if _TAIL and b == _NLEFT:
                continue
            store_wait(b)

    return k(sig_table, idx)


def kernel(X, Y, gate_theta):
    sig_table = _sigmoid_table(gate_theta)
    idx = Y.astype(jnp.int32)
    return _gather(sig_table, idx)
